# baseline (speedup 1.0000x reference)
import functools

import jax
import jax.numpy as jnp
from jax import lax
from jax.experimental import pallas as pl
from jax.experimental.pallas import tpu as pltpu
from jax.experimental.pallas import tpu_sc as plsc

N0, N1, N2 = 50000, 20000, 5000
E0, E1 = 320000, 80000
D, DH = 128, 64

NC, NS, LANES = 2, 16, 16
CH = 128

C0 = 160
IB = 40
T0E = C0 * CH
EP0 = NS * T0E
R1P = 20480

C1 = 40
T1E = C1 * CH
EP1 = NS * T1E
R2P = 5120


def _pad_edges(src, dst, ep, n_src, n_dst, n_dst_pad):
    pe = ep - src.shape[0]
    ar = jnp.arange(pe, dtype=jnp.int32)
    src_p = jnp.concatenate([src, ar % n_src])
    dst_p = jnp.concatenate([dst, n_dst + ar % (n_dst_pad - n_dst)])
    return src_p, dst_p


def _sc_layer0_body(h_ref, src_ref, dst_ref, dst1_ref, agg_out,
                    deg0_out, deg1_out,
                    src_v, dst_v, rows0_v, rows1_v, rows2_v, rows3_v, ones_v,
                    zb16_v, acc_sp, deg0a_sp, deg0b_sp, deg1_sp,
                    g0, g1, g2, g3, s0, s1, s2, s3, dsem):
    c = lax.axis_index("c")
    s = lax.axis_index("s")
    zv = jnp.zeros((2 * LANES,), jnp.bfloat16)
    zf = jnp.zeros((LANES,), jnp.float32)
    of = jnp.ones((LANES,), jnp.float32)
    rows = (rows0_v, rows1_v, rows2_v, rows3_v)
    gsem = (g0, g1, g2, g3)
    ssem = (s0, s1, s2, s3)

    def zero_rows0(i, _):
        r, q = i // 2, i % 2
        rows0_v[r, pl.ds(q * 2 * LANES, 2 * LANES)] = zv
        return 0

    def fill16(i, _):
        ones_v[i, :] = of
        zb16_v[i, :] = zf
        return 0

    def zero_acc():
        lax.fori_loop(0, 256, zero_rows0, 0)
        for k in range(10):
            pltpu.sync_copy(rows0_v, acc_sp.at[pl.ds(base + k * CH, CH)])

    base = s * (R1P // NS)
    base1 = s * (R2P // NS)
    zero_acc()
    lax.fori_loop(0, CH, fill16, 0)
    deg0_sps = (deg0a_sp, deg0b_sp)
    for rr in range(2):
        for k in range(10):
            pltpu.sync_copy(zb16_v,
                            deg0_sps[rr].at[pl.ds(base + k * CH, CH)])
    pltpu.sync_copy(zb16_v, deg1_sp.at[pl.ds(base1, CH)])
    pltpu.sync_copy(zb16_v, deg1_sp.at[pl.ds(base1 + CH, CH)])
    pltpu.sync_copy(zb16_v.at[:64], deg1_sp.at[pl.ds(base1 + 2 * CH, 64)])
    plsc.subcore_barrier()

    nblk = C0 // IB
    for r in range(2):
        for hb in range(nblk):
            p_deg = (c == 0) == (hb < nblk // 2)
            pltpu.sync_copy(src_ref.at[c, r, s, pl.ds(hb * IB, IB)], src_v)
            pltpu.sync_copy(dst_ref.at[r, s, pl.ds(hb * IB, IB)], dst_v)

            for k in range(4):
                pltpu.async_copy(h_ref.at[src_v.at[k]], rows[k], gsem[k])

            def quad(qq, _):
                j = 4 * qq
                for k in range(4):
                    pltpu.make_async_copy(h_ref.at[src_v.at[j + k]], rows[k],
                                          gsem[k]).wait()
                    pltpu.async_copy(rows[k], acc_sp.at[dst_v.at[j + k]],
                                     ssem[k], add=True)

                    @pl.when(p_deg & (j + k >= 8))
                    def _():
                        pltpu.make_async_copy(ones_v,
                                              deg0_sps[r].at[dst_v.at[0]],
                                              dsem).wait()

                    @pl.when(p_deg)
                    def _():
                        pltpu.async_copy(ones_v,
                                         deg0_sps[r].at[dst_v.at[j + k]],
                                         dsem, add=True)

                @pl.when(qq < IB // 4 - 1)
                def _():
                    for k in range(4):
                        pltpu.make_async_copy(rows[k],
                                              acc_sp.at[dst_v.at[j + k]],
                                              ssem[k]).wait()
                        pltpu.async_copy(h_ref.at[src_v.at[j + 4 + k]],
                                         rows[k], gsem[k])
                return 0
            lax.fori_loop(0, IB // 4, quad, 0)
            for k in range(4):
                pltpu.make_async_copy(rows[k], acc_sp.at[dst_v.at[k]],
                                      ssem[k]).wait()

            @pl.when(p_deg)
            def _():
                for _k in range(8):
                    pltpu.make_async_copy(ones_v,
                                          deg0_sps[r].at[dst_v.at[0]],
                                          dsem).wait()
        plsc.subcore_barrier()

        @pl.when(c == 0)
        def _():
            pltpu.sync_copy(acc_sp.at[pl.ds(base, R1P // NS)],
                            agg_out.at[r, pl.ds(base, R1P // NS),
                                       pl.ds(0, DH)])

        @pl.when(c == 1)
        def _():
            pltpu.sync_copy(acc_sp.at[pl.ds(base, R1P // NS)],
                            agg_out.at[r, pl.ds(base, R1P // NS),
                                       pl.ds(DH, DH)])
        if r == 0:
            zero_acc()
            plsc.subcore_barrier()

    pltpu.sync_copy(dst1_ref.at[c, s], dst_v.at[pl.ds(0, C1)])

    def dchunk(j, _):
        @pl.when(j >= 8)
        def _():
            pltpu.make_async_copy(ones_v, deg1_sp.at[dst_v.at[0]],
                                  dsem).wait()
        pltpu.async_copy(ones_v, deg1_sp.at[dst_v.at[j]], dsem, add=True)
        return 0
    lax.fori_loop(0, C1, dchunk, 0)
    for _k in range(8):
        pltpu.make_async_copy(ones_v, deg1_sp.at[dst_v.at[0]], dsem).wait()
    plsc.subcore_barrier()

    for rr in range(2):
        @pl.when(c == 0)
        def _():
            pltpu.sync_copy(deg0_sps[rr].at[pl.ds(base, R1P // NS)],
                            deg0_out.at[rr, pl.ds(base, R1P // NS),
                                        pl.ds(0, LANES)])

        @pl.when(c == 1)
        def _():
            pltpu.sync_copy(deg0_sps[rr].at[pl.ds(base, R1P // NS)],
                            deg0_out.at[rr, pl.ds(base, R1P // NS),
                                        pl.ds(LANES, LANES)])
    pltpu.sync_copy(deg1_sp.at[pl.ds(base1, R2P // NS)],
                    deg1_out.at[c, pl.ds(base1, R2P // NS), pl.ds(0, LANES)])


_sc_layer0 = functools.partial(
    pl.kernel,
    out_type=(jax.ShapeDtypeStruct((2, R1P, D), jnp.bfloat16),
              jax.ShapeDtypeStruct((2, R1P, D), jnp.float32),
              jax.ShapeDtypeStruct((2, R2P, D), jnp.float32)),
    mesh=plsc.VectorSubcoreMesh(core_axis_name="c", subcore_axis_name="s"),
    compiler_params=pltpu.CompilerParams(use_tc_tiling_on_sc=False),
    scratch_types=[
        pltpu.VMEM((IB, CH), jnp.int32),
        pltpu.VMEM((IB, CH), jnp.int32),
        pltpu.VMEM((CH, DH), jnp.bfloat16),
        pltpu.VMEM((CH, DH), jnp.bfloat16),
        pltpu.VMEM((CH, DH), jnp.bfloat16),
        pltpu.VMEM((CH, DH), jnp.bfloat16),
        pltpu.VMEM((CH, LANES), jnp.float32),
        pltpu.VMEM((CH, LANES), jnp.float32),
        pltpu.VMEM_SHARED((R1P, DH), jnp.bfloat16),
        pltpu.VMEM_SHARED((R1P, LANES), jnp.float32),
        pltpu.VMEM_SHARED((R1P, LANES), jnp.float32),
        pltpu.VMEM_SHARED((R2P, LANES), jnp.float32),
    ] + [pltpu.SemaphoreType.DMA] * 9,
)(_sc_layer0_body)


BR = 512


def _tc_dense_body(hd_ref, agg_ref, deg_ref, ws_ref, wn_ref, b_ref, wcat_ref,
                   q_ref, p_ref):
    hd = hd_ref[...]
    a = agg_ref[...]
    d = deg_ref[...]
    deg0 = jnp.maximum(d[0, :, 0:1] + d[0, :, LANES:LANES + 1], 1.0)
    deg1 = jnp.maximum(d[1, :, 0:1] + d[1, :, LANES:LANES + 1], 1.0)
    m0 = a[0].astype(jnp.float32) / deg0
    m1 = a[1].astype(jnp.float32) / deg1
    o = (jax.nn.relu(jnp.dot(hd, ws_ref[0], preferred_element_type=jnp.float32)
                     + jnp.dot(m0, wn_ref[0], preferred_element_type=jnp.float32)
                     + b_ref[0])
         + jax.nn.relu(jnp.dot(hd, ws_ref[1], preferred_element_type=jnp.float32)
                       + jnp.dot(m1, wn_ref[1], preferred_element_type=jnp.float32)
                       + b_ref[1]))
    qp = jnp.dot(o, wcat_ref[...], preferred_element_type=jnp.float32)
    q_ref[0] = jnp.broadcast_to(qp[:, 0:1], (BR, LANES))
    q_ref[1] = jnp.broadcast_to(qp[:, 1:2], (BR, LANES))
    p_ref[...] = jnp.broadcast_to(qp[:, 2:3], (BR, LANES))


def _tc_dense(hd, agg, deg, ws, wn, b2, wcat):
    nb = R1P // BR
    return pl.pallas_call(
        _tc_dense_body,
        grid=(nb,),
        in_specs=[
            pl.BlockSpec((BR, D), lambda i: (i, 0)),
            pl.BlockSpec((2, BR, D), lambda i: (0, i, 0)),
            pl.BlockSpec((2, BR, D), lambda i: (0, i, 0)),
            pl.BlockSpec((2, D, D), lambda i: (0, 0, 0)),
            pl.BlockSpec((2, D, D), lambda i: (0, 0, 0)),
            pl.BlockSpec((2, 1, D), lambda i: (0, 0, 0)),
            pl.BlockSpec((D, D), lambda i: (0, 0)),
        ],
        out_specs=[
            pl.BlockSpec((2, BR, LANES), lambda i: (0, i, 0)),
            pl.BlockSpec((BR, LANES), lambda i: (i, 0)),
        ],
        out_shape=[
            jax.ShapeDtypeStruct((2, R1P, LANES), jnp.float32),
            jax.ShapeDtypeStruct((R1P, LANES), jnp.float32),
        ],
    )(hd, agg, deg, ws, wn, b2, wcat)


def _sc_layer1_body(q_ref, src_ref, dst_ref, sd_out,
                    src_v, dst_v, rows0_v, rows1_v, zb16_v, s_sp,
                    gsem0, gsem1, ssem0, ssem1):
    c = lax.axis_index("c")
    s = lax.axis_index("s")
    zv = jnp.zeros((LANES,), jnp.float32)

    def fill16(i, _):
        zb16_v[i, :] = zv
        return 0
    lax.fori_loop(0, CH, fill16, 0)

    base = s * (R2P // NS)
    pltpu.sync_copy(zb16_v, s_sp.at[pl.ds(base, CH)])
    pltpu.sync_copy(zb16_v, s_sp.at[pl.ds(base + CH, CH)])
    pltpu.sync_copy(zb16_v.at[:64], s_sp.at[pl.ds(base + 2 * CH, 64)])
    plsc.subcore_barrier()

    pltpu.sync_copy(src_ref.at[c, s], src_v)
    pltpu.sync_copy(dst_ref.at[c, s], dst_v)

    pltpu.async_copy(q_ref.at[src_v.at[0]], rows0_v, gsem0)
    pltpu.async_copy(q_ref.at[src_v.at[1]], rows1_v, gsem1)

    def pair(jj, _):
        j0 = 2 * jj
        pltpu.make_async_copy(q_ref.at[src_v.at[j0]], rows0_v, gsem0).wait()
        pltpu.async_copy(rows0_v, s_sp.at[dst_v.at[j0]], ssem0, add=True)
        pltpu.make_async_copy(q_ref.at[src_v.at[j0 + 1]], rows1_v,
                              gsem1).wait()
        pltpu.async_copy(rows1_v, s_sp.at[dst_v.at[j0 + 1]], ssem1, add=True)

        @pl.when(jj < C1 // 2 - 1)
        def _():
            pltpu.make_async_copy(rows0_v, s_sp.at[dst_v.at[j0]],
                                  ssem0).wait()
            pltpu.async_copy(q_ref.at[src_v.at[j0 + 2]], rows0_v, gsem0)
            pltpu.make_async_copy(rows1_v, s_sp.at[dst_v.at[j0 + 1]],
                                  ssem1).wait()
            pltpu.async_copy(q_ref.at[src_v.at[j0 + 3]], rows1_v, gsem1)
        return 0
    lax.fori_loop(0, C1 // 2, pair, 0)
    pltpu.make_async_copy(rows0_v, s_sp.at[dst_v.at[0]], ssem0).wait()
    pltpu.make_async_copy(rows1_v, s_sp.at[dst_v.at[1]], ssem1).wait()
    plsc.subcore_barrier()

    nrow = R2P // NS

    @pl.when(c == 0)
    def _():
        pltpu.sync_copy(s_sp.at[pl.ds(base, nrow)],
                        sd_out.at[pl.ds(base, nrow), pl.ds(0, LANES)])

    @pl.when(c == 1)
    def _():
        pltpu.sync_copy(s_sp.at[pl.ds(base, nrow)],
                        sd_out.at[pl.ds(base, nrow), pl.ds(LANES, LANES)])


_sc_layer1 = functools.partial(
    pl.kernel,
    out_type=jax.ShapeDtypeStruct((R2P, D), jnp.float32),
    mesh=plsc.VectorSubcoreMesh(core_axis_name="c", subcore_axis_name="s"),
    compiler_params=pltpu.CompilerParams(use_tc_tiling_on_sc=False),
    scratch_types=[
        pltpu.VMEM((C1, CH), jnp.int32),
        pltpu.VMEM((C1, CH), jnp.int32),
        pltpu.VMEM((CH, LANES), jnp.float32),
        pltpu.VMEM((CH, LANES), jnp.float32),
        pltpu.VMEM((CH, LANES), jnp.float32),
        pltpu.VMEM_SHARED((R2P, LANES), jnp.float32),
        pltpu.SemaphoreType.DMA,
        pltpu.SemaphoreType.DMA,
        pltpu.SemaphoreType.DMA,
        pltpu.SemaphoreType.DMA,
    ],
)(_sc_layer1_body)


def _tc_final_body(p_ref, sd_ref, dg_ref, b_ref, o_ref):
    p = p_ref[:, 0:1]
    sd = sd_ref[...]
    dg = dg_ref[...]
    s0 = sd[:, 0:1] / jnp.maximum(dg[0, :, 0:1], 1.0)
    s1 = sd[:, LANES:LANES + 1] / jnp.maximum(dg[1, :, 0:1], 1.0)
    o_ref[...] = jnp.broadcast_to(
        jax.nn.sigmoid(p + s0 + s1 + b_ref[0, 0:1]), (R2P, LANES))


def _tc_final(p, sd, deg1, bsum):
    return pl.pallas_call(
        _tc_final_body,
        grid=(1,),
        in_specs=[
            pl.BlockSpec((R2P, LANES), lambda i: (0, 0)),
            pl.BlockSpec((R2P, D), lambda i: (0, 0)),
            pl.BlockSpec((2, R2P, D), lambda i: (0, 0, 0)),
            pl.BlockSpec((1, LANES), lambda i: (0, 0)),
        ],
        out_specs=pl.BlockSpec((R2P, LANES), lambda i: (0, 0)),
        out_shape=jax.ShapeDtypeStruct((R2P, LANES), jnp.float32),
    )(p, sd, deg1, bsum)


def kernel(h, src0_r0, dst0_r0, src0_r1, dst0_r1, src1_r0, dst1_r0,
           src1_r1, dst1_r1, Wself0_r0, Wneigh0_r0, b0_r0, Wself0_r1,
           Wneigh0_r1, b0_r1, Wself1_r0, Wneigh1_r0, b1_r0, Wself1_r1,
           Wneigh1_r1, b1_r1):
    hflat = h.astype(jnp.bfloat16).reshape(2 * N0, DH)

    s0p, d0p = _pad_edges(src0_r0, dst0_r0, EP0, N0, N1, R1P)
    s1p, d1p = _pad_edges(src0_r1, dst0_r1, EP0, N0, N1, R1P)
    srcA = jnp.stack([
        jnp.stack([(2 * s0p).reshape(NS, C0, CH),
                   (2 * s1p).reshape(NS, C0, CH)]),
        jnp.stack([(2 * s0p + 1).reshape(NS, C0, CH),
                   (2 * s1p + 1).reshape(NS, C0, CH)]),
    ])
    dstA = jnp.stack([d0p.reshape(NS, C0, CH), d1p.reshape(NS, C0, CH)])

    sb0, db0 = _pad_edges(src1_r0, dst1_r0, EP1, N1, N2, R2P)
    sb1, db1 = _pad_edges(src1_r1, dst1_r1, EP1, N1, N2, R2P)
    srcB = jnp.stack([sb0.reshape(NS, C1, CH),
                      (sb1 + R1P).reshape(NS, C1, CH)])
    dstB = jnp.stack([db0.reshape(NS, C1, CH), db1.reshape(NS, C1, CH)])

    agg, deg0, deg1 = _sc_layer0(hflat, srcA, dstA, dstB)

    ws = jnp.stack([Wself0_r0, Wself0_r1])
    wn = jnp.stack([Wneigh0_r0, Wneigh0_r1])
    b2 = jnp.stack([b0_r0, b0_r1]).reshape(2, 1, D)
    wcat = jnp.concatenate(
        [Wneigh1_r0, Wneigh1_r1, Wself1_r0 + Wself1_r1,
         jnp.zeros((D, D - 3), jnp.float32)], axis=1)

    q2, p = _tc_dense(h, agg, deg0, ws, wn, b2, wcat)
    qf = q2.reshape(2 * R1P, LANES)

    sd = _sc_layer1(qf, srcB, dstB)

    bsum = jnp.broadcast_to((b1_r0 + b1_r1).reshape(1, 1), (1, LANES))
    out = _tc_final(p, sd, deg1, bsum)
    return out[:N2, 0:1]


# speedup vs baseline: 1.0882x; 1.0882x over previous
import functools

import jax
import jax.numpy as jnp
from jax import lax
from jax.experimental import pallas as pl
from jax.experimental.pallas import tpu as pltpu
from jax.experimental.pallas import tpu_sc as plsc

N0, N1, N2 = 50000, 20000, 5000
E0, E1 = 320000, 80000
D, DH = 128, 64

NC, NS, LANES = 2, 16, 16
CH = 128

C0 = 160
IB = 40
T0E = C0 * CH
EP0 = NS * T0E
R1P = 20480

C1 = 40
T1E = C1 * CH
EP1 = NS * T1E
R2P = 5120


def _pad_edges(src, dst, ep, n_src, n_dst, n_dst_pad):
    pe = ep - src.shape[0]
    ar = jnp.arange(pe, dtype=jnp.int32)
    src_p = jnp.concatenate([src, ar % n_src])
    dst_p = jnp.concatenate([dst, n_dst + ar % (n_dst_pad - n_dst)])
    return src_p, dst_p


def _sc_layer0_body(h_ref, src_ref, dst_ref, agg_out,
                    src_v, dst_v, rows0_v, rows1_v, rows2_v, rows3_v, acc_sp,
                    g0, g1, g2, g3, s0, s1, s2, s3):
    c = lax.axis_index("c")
    s = lax.axis_index("s")
    zv = jnp.zeros((2 * LANES,), jnp.bfloat16)
    rows = (rows0_v, rows1_v, rows2_v, rows3_v)
    gsem = (g0, g1, g2, g3)
    ssem = (s0, s1, s2, s3)

    def zero_rows0(i, _):
        r, q = i // 2, i % 2
        rows0_v[r, pl.ds(q * 2 * LANES, 2 * LANES)] = zv
        return 0

    def zero_acc():
        lax.fori_loop(0, 256, zero_rows0, 0)
        for k in range(10):
            pltpu.sync_copy(rows0_v, acc_sp.at[pl.ds(base + k * CH, CH)])

    base = s * (R1P // NS)
    zero_acc()
    plsc.subcore_barrier()

    nblk = C0 // IB
    for r in range(2):
        for hb in range(nblk):
            pltpu.sync_copy(src_ref.at[c, r, s, pl.ds(hb * IB, IB)], src_v)
            pltpu.sync_copy(dst_ref.at[r, s, pl.ds(hb * IB, IB)], dst_v)

            for k in range(4):
                pltpu.async_copy(h_ref.at[src_v.at[k]], rows[k], gsem[k])

            def quad(qq, _):
                j = 4 * qq
                for k in range(4):
                    pltpu.make_async_copy(h_ref.at[src_v.at[j + k]], rows[k],
                                          gsem[k]).wait()
                    pltpu.async_copy(rows[k], acc_sp.at[dst_v.at[j + k]],
                                     ssem[k], add=True)

                @pl.when(qq < IB // 4 - 1)
                def _():
                    for k in range(4):
                        pltpu.make_async_copy(rows[k],
                                              acc_sp.at[dst_v.at[j + k]],
                                              ssem[k]).wait()
                        pltpu.async_copy(h_ref.at[src_v.at[j + 4 + k]],
                                         rows[k], gsem[k])
                return 0
            lax.fori_loop(0, IB // 4, quad, 0)
            for k in range(4):
                pltpu.make_async_copy(rows[k], acc_sp.at[dst_v.at[k]],
                                      ssem[k]).wait()
        plsc.subcore_barrier()

        @pl.when(c == 0)
        def _():
            pltpu.sync_copy(acc_sp.at[pl.ds(base, R1P // NS)],
                            agg_out.at[r, pl.ds(base, R1P // NS),
                                       pl.ds(0, DH)])

        @pl.when(c == 1)
        def _():
            pltpu.sync_copy(acc_sp.at[pl.ds(base, R1P // NS)],
                            agg_out.at[r, pl.ds(base, R1P // NS),
                                       pl.ds(DH, DH)])
        if r == 0:
            zero_acc()
            plsc.subcore_barrier()


_sc_layer0 = functools.partial(
    pl.kernel,
    out_type=jax.ShapeDtypeStruct((2, R1P, D), jnp.bfloat16),
    mesh=plsc.VectorSubcoreMesh(core_axis_name="c", subcore_axis_name="s"),
    compiler_params=pltpu.CompilerParams(use_tc_tiling_on_sc=False),
    scratch_types=[
        pltpu.VMEM((IB, CH), jnp.int32),
        pltpu.VMEM((IB, CH), jnp.int32),
        pltpu.VMEM((CH, DH), jnp.bfloat16),
        pltpu.VMEM((CH, DH), jnp.bfloat16),
        pltpu.VMEM((CH, DH), jnp.bfloat16),
        pltpu.VMEM((CH, DH), jnp.bfloat16),
        pltpu.VMEM_SHARED((R1P, DH), jnp.bfloat16),
    ] + [pltpu.SemaphoreType.DMA] * 8,
)(_sc_layer0_body)


def _sc_degrees_body(dst0_ref, dst1_ref, deg0_out, deg1_out,
                     dst_v, ones_v, zb16_v, deg0_sp, deg1_sp, dsem):
    c = lax.axis_index("c")
    s = lax.axis_index("s")
    zv = jnp.zeros((LANES,), jnp.float32)
    ov = jnp.ones((LANES,), jnp.float32)

    def fill16(i, _):
        ones_v[i, :] = ov
        zb16_v[i, :] = zv
        return 0
    lax.fori_loop(0, CH, fill16, 0)

    base0 = s * (R1P // NS)
    base1 = s * (R2P // NS)
    for k in range(10):
        pltpu.sync_copy(zb16_v, deg0_sp.at[pl.ds(base0 + k * CH, CH)])
    pltpu.sync_copy(zb16_v, deg1_sp.at[pl.ds(base1, CH)])
    pltpu.sync_copy(zb16_v, deg1_sp.at[pl.ds(base1 + CH, CH)])
    pltpu.sync_copy(zb16_v.at[:64], deg1_sp.at[pl.ds(base1 + 2 * CH, 64)])
    plsc.subcore_barrier()

    pltpu.sync_copy(dst0_ref.at[c, s], dst_v)

    def chunk0(j, _):
        @pl.when(j >= 8)
        def _():
            pltpu.make_async_copy(ones_v, deg0_sp.at[dst_v.at[0]],
                                  dsem).wait()
        pltpu.async_copy(ones_v, deg0_sp.at[dst_v.at[j]], dsem, add=True)
        return 0
    lax.fori_loop(0, C0, chunk0, 0)
    for _ in range(8):
        pltpu.make_async_copy(ones_v, deg0_sp.at[dst_v.at[0]], dsem).wait()

    pltpu.sync_copy(dst1_ref.at[c, s], dst_v.at[pl.ds(0, C1)])

    def chunk1(j, _):
        @pl.when(j >= 8)
        def _():
            pltpu.make_async_copy(ones_v, deg1_sp.at[dst_v.at[0]],
                                  dsem).wait()
        pltpu.async_copy(ones_v, deg1_sp.at[dst_v.at[j]], dsem, add=True)
        return 0
    lax.fori_loop(0, C1, chunk1, 0)
    for _ in range(8):
        pltpu.make_async_copy(ones_v, deg1_sp.at[dst_v.at[0]], dsem).wait()
    plsc.subcore_barrier()

    pltpu.sync_copy(deg0_sp.at[pl.ds(base0, R1P // NS)],
                    deg0_out.at[c, pl.ds(base0, R1P // NS), pl.ds(0, LANES)])
    pltpu.sync_copy(deg1_sp.at[pl.ds(base1, R2P // NS)],
                    deg1_out.at[c, pl.ds(base1, R2P // NS), pl.ds(0, LANES)])


_sc_degrees = functools.partial(
    pl.kernel,
    out_type=(jax.ShapeDtypeStruct((2, R1P, D), jnp.float32),
              jax.ShapeDtypeStruct((2, R2P, D), jnp.float32)),
    mesh=plsc.VectorSubcoreMesh(core_axis_name="c", subcore_axis_name="s"),
    compiler_params=pltpu.CompilerParams(use_tc_tiling_on_sc=False),
    scratch_types=[
        pltpu.VMEM((C0, CH), jnp.int32),
        pltpu.VMEM((CH, LANES), jnp.float32),
        pltpu.VMEM((CH, LANES), jnp.float32),
        pltpu.VMEM_SHARED((R1P, LANES), jnp.float32),
        pltpu.VMEM_SHARED((R2P, LANES), jnp.float32),
        pltpu.SemaphoreType.DMA,
    ],
)(_sc_degrees_body)


BR = 512


def _tc_dense_body(hd_ref, agg_ref, deg_ref, ws_ref, wn_ref, b_ref, wcat_ref,
                   q_ref, p_ref):
    hd = hd_ref[...]
    a = agg_ref[...]
    d = deg_ref[...]
    deg0 = jnp.maximum(d[0, :, 0:1], 1.0)
    deg1 = jnp.maximum(d[1, :, 0:1], 1.0)
    m0 = a[0].astype(jnp.float32) / deg0
    m1 = a[1].astype(jnp.float32) / deg1
    o = (jax.nn.relu(jnp.dot(hd, ws_ref[0], preferred_element_type=jnp.float32)
                     + jnp.dot(m0, wn_ref[0], preferred_element_type=jnp.float32)
                     + b_ref[0])
         + jax.nn.relu(jnp.dot(hd, ws_ref[1], preferred_element_type=jnp.float32)
                       + jnp.dot(m1, wn_ref[1], preferred_element_type=jnp.float32)
                       + b_ref[1]))
    qp = jnp.dot(o, wcat_ref[...], preferred_element_type=jnp.float32)
    q_ref[0] = jnp.broadcast_to(qp[:, 0:1], (BR, LANES))
    q_ref[1] = jnp.broadcast_to(qp[:, 1:2], (BR, LANES))
    p_ref[...] = jnp.broadcast_to(qp[:, 2:3], (BR, LANES))


def _tc_dense(hd, agg, deg, ws, wn, b2, wcat):
    nb = R1P // BR
    return pl.pallas_call(
        _tc_dense_body,
        grid=(nb,),
        in_specs=[
            pl.BlockSpec((BR, D), lambda i: (i, 0)),
            pl.BlockSpec((2, BR, D), lambda i: (0, i, 0)),
            pl.BlockSpec((2, BR, D), lambda i: (0, i, 0)),
            pl.BlockSpec((2, D, D), lambda i: (0, 0, 0)),
            pl.BlockSpec((2, D, D), lambda i: (0, 0, 0)),
            pl.BlockSpec((2, 1, D), lambda i: (0, 0, 0)),
            pl.BlockSpec((D, D), lambda i: (0, 0)),
        ],
        out_specs=[
            pl.BlockSpec((2, BR, LANES), lambda i: (0, i, 0)),
            pl.BlockSpec((BR, LANES), lambda i: (i, 0)),
        ],
        out_shape=[
            jax.ShapeDtypeStruct((2, R1P, LANES), jnp.float32),
            jax.ShapeDtypeStruct((R1P, LANES), jnp.float32),
        ],
    )(hd, agg, deg, ws, wn, b2, wcat)


def _sc_layer1_body(q_ref, src_ref, dst_ref, sd_out,
                    src_v, dst_v, rows0_v, rows1_v, zb16_v, s_sp,
                    gsem0, gsem1, ssem0, ssem1):
    c = lax.axis_index("c")
    s = lax.axis_index("s")
    zv = jnp.zeros((LANES,), jnp.float32)

    def fill16(i, _):
        zb16_v[i, :] = zv
        return 0
    lax.fori_loop(0, CH, fill16, 0)

    base = s * (R2P // NS)
    pltpu.sync_copy(zb16_v, s_sp.at[pl.ds(base, CH)])
    pltpu.sync_copy(zb16_v, s_sp.at[pl.ds(base + CH, CH)])
    pltpu.sync_copy(zb16_v.at[:64], s_sp.at[pl.ds(base + 2 * CH, 64)])
    plsc.subcore_barrier()

    pltpu.sync_copy(src_ref.at[c, s], src_v)
    pltpu.sync_copy(dst_ref.at[c, s], dst_v)

    pltpu.async_copy(q_ref.at[src_v.at[0]], rows0_v, gsem0)
    pltpu.async_copy(q_ref.at[src_v.at[1]], rows1_v, gsem1)

    def pair(jj, _):
        j0 = 2 * jj
        pltpu.make_async_copy(q_ref.at[src_v.at[j0]], rows0_v, gsem0).wait()
        pltpu.async_copy(rows0_v, s_sp.at[dst_v.at[j0]], ssem0, add=True)
        pltpu.make_async_copy(q_ref.at[src_v.at[j0 + 1]], rows1_v,
                              gsem1).wait()
        pltpu.async_copy(rows1_v, s_sp.at[dst_v.at[j0 + 1]], ssem1, add=True)

        @pl.when(jj < C1 // 2 - 1)
        def _():
            pltpu.make_async_copy(rows0_v, s_sp.at[dst_v.at[j0]],
                                  ssem0).wait()
            pltpu.async_copy(q_ref.at[src_v.at[j0 + 2]], rows0_v, gsem0)
            pltpu.make_async_copy(rows1_v, s_sp.at[dst_v.at[j0 + 1]],
                                  ssem1).wait()
            pltpu.async_copy(q_ref.at[src_v.at[j0 + 3]], rows1_v, gsem1)
        return 0
    lax.fori_loop(0, C1 // 2, pair, 0)
    pltpu.make_async_copy(rows0_v, s_sp.at[dst_v.at[0]], ssem0).wait()
    pltpu.make_async_copy(rows1_v, s_sp.at[dst_v.at[1]], ssem1).wait()
    plsc.subcore_barrier()

    nrow = R2P // NS

    @pl.when(c == 0)
    def _():
        pltpu.sync_copy(s_sp.at[pl.ds(base, nrow)],
                        sd_out.at[pl.ds(base, nrow), pl.ds(0, LANES)])

    @pl.when(c == 1)
    def _():
        pltpu.sync_copy(s_sp.at[pl.ds(base, nrow)],
                        sd_out.at[pl.ds(base, nrow), pl.ds(LANES, LANES)])


_sc_layer1 = functools.partial(
    pl.kernel,
    out_type=jax.ShapeDtypeStruct((R2P, D), jnp.float32),
    mesh=plsc.VectorSubcoreMesh(core_axis_name="c", subcore_axis_name="s"),
    compiler_params=pltpu.CompilerParams(use_tc_tiling_on_sc=False),
    scratch_types=[
        pltpu.VMEM((C1, CH), jnp.int32),
        pltpu.VMEM((C1, CH), jnp.int32),
        pltpu.VMEM((CH, LANES), jnp.float32),
        pltpu.VMEM((CH, LANES), jnp.float32),
        pltpu.VMEM((CH, LANES), jnp.float32),
        pltpu.VMEM_SHARED((R2P, LANES), jnp.float32),
        pltpu.SemaphoreType.DMA,
        pltpu.SemaphoreType.DMA,
        pltpu.SemaphoreType.DMA,
        pltpu.SemaphoreType.DMA,
    ],
)(_sc_layer1_body)


def _tc_final_body(p_ref, sd_ref, dg_ref, b_ref, o_ref):
    p = p_ref[:, 0:1]
    sd = sd_ref[...]
    dg = dg_ref[...]
    s0 = sd[:, 0:1] / jnp.maximum(dg[0, :, 0:1], 1.0)
    s1 = sd[:, LANES:LANES + 1] / jnp.maximum(dg[1, :, 0:1], 1.0)
    o_ref[...] = jnp.broadcast_to(
        jax.nn.sigmoid(p + s0 + s1 + b_ref[0, 0:1]), (R2P, LANES))


def _tc_final(p, sd, deg1, bsum):
    return pl.pallas_call(
        _tc_final_body,
        grid=(1,),
        in_specs=[
            pl.BlockSpec((R2P, LANES), lambda i: (0, 0)),
            pl.BlockSpec((R2P, D), lambda i: (0, 0)),
            pl.BlockSpec((2, R2P, D), lambda i: (0, 0, 0)),
            pl.BlockSpec((1, LANES), lambda i: (0, 0)),
        ],
        out_specs=pl.BlockSpec((R2P, LANES), lambda i: (0, 0)),
        out_shape=jax.ShapeDtypeStruct((R2P, LANES), jnp.float32),
    )(p, sd, deg1, bsum)


def kernel(h, src0_r0, dst0_r0, src0_r1, dst0_r1, src1_r0, dst1_r0,
           src1_r1, dst1_r1, Wself0_r0, Wneigh0_r0, b0_r0, Wself0_r1,
           Wneigh0_r1, b0_r1, Wself1_r0, Wneigh1_r0, b1_r0, Wself1_r1,
           Wneigh1_r1, b1_r1):
    hflat = h.astype(jnp.bfloat16).reshape(2 * N0, DH)

    s0p, d0p = _pad_edges(src0_r0, dst0_r0, EP0, N0, N1, R1P)
    s1p, d1p = _pad_edges(src0_r1, dst0_r1, EP0, N0, N1, R1P)
    srcA = jnp.stack([
        jnp.stack([(2 * s0p).reshape(NS, C0, CH),
                   (2 * s1p).reshape(NS, C0, CH)]),
        jnp.stack([(2 * s0p + 1).reshape(NS, C0, CH),
                   (2 * s1p + 1).reshape(NS, C0, CH)]),
    ])
    dstA = jnp.stack([d0p.reshape(NS, C0, CH), d1p.reshape(NS, C0, CH)])

    agg = _sc_layer0(hflat, srcA, dstA)

    ws = jnp.stack([Wself0_r0, Wself0_r1])
    wn = jnp.stack([Wneigh0_r0, Wneigh0_r1])
    b2 = jnp.stack([b0_r0, b0_r1]).reshape(2, 1, D)
    wcat = jnp.concatenate(
        [Wneigh1_r0, Wneigh1_r1, Wself1_r0 + Wself1_r1,
         jnp.zeros((D, D - 3), jnp.float32)], axis=1)

    sb0, db0 = _pad_edges(src1_r0, dst1_r0, EP1, N1, N2, R2P)
    sb1, db1 = _pad_edges(src1_r1, dst1_r1, EP1, N1, N2, R2P)
    srcB = jnp.stack([sb0.reshape(NS, C1, CH),
                      (sb1 + R1P).reshape(NS, C1, CH)])
    dstB = jnp.stack([db0.reshape(NS, C1, CH), db1.reshape(NS, C1, CH)])

    deg0, deg1 = _sc_degrees(dstA, dstB)

    q2, p = _tc_dense(h, agg, deg0, ws, wn, b2, wcat)
    qf = q2.reshape(2 * R1P, LANES)

    sd = _sc_layer1(qf, srcB, dstB)

    bsum = jnp.broadcast_to((b1_r0 + b1_r1).reshape(1, 1), (1, LANES))
    out = _tc_final(p, sd, deg1, bsum)
    return out[:N2, 0:1]


# speedup vs baseline: 1.1332x; 1.0413x over previous
import functools

import jax
import jax.numpy as jnp
from jax import lax
from jax.experimental import pallas as pl
from jax.experimental.pallas import tpu as pltpu
from jax.experimental.pallas import tpu_sc as plsc

N0, N1, N2 = 50000, 20000, 5000
E0, E1 = 320000, 80000
D, DH = 128, 64

NC, NS, LANES = 2, 16, 16
CH = 128

C0 = 160
IB = 40
T0E = C0 * CH
EP0 = NS * T0E
R1P = 20480

C1 = 40
T1E = C1 * CH
EP1 = NS * T1E
R2P = 5120


def _pad_edges(src, dst, ep, n_src, n_dst, n_dst_pad):
    pe = ep - src.shape[0]
    ar = jnp.arange(pe, dtype=jnp.int32)
    src_p = jnp.concatenate([src, ar % n_src])
    dst_p = jnp.concatenate([dst, n_dst + ar % (n_dst_pad - n_dst)])
    return src_p, dst_p


def _sc_layer0_body(h_ref, src_ref, dst_ref, agg_out,
                    src_v, dst_v, rows0_v, rows1_v, rows2_v, rows3_v, acc_sp,
                    g0, g1, g2, g3, s0, s1, s2, s3):
    c = lax.axis_index("c")
    s = lax.axis_index("s")
    zv = jnp.zeros((2 * LANES,), jnp.bfloat16)
    rows = (rows0_v, rows1_v, rows2_v, rows3_v)
    gsem = (g0, g1, g2, g3)
    ssem = (s0, s1, s2, s3)

    def zero_rows0(i, _):
        r, q = i // 2, i % 2
        rows0_v[r, pl.ds(q * 2 * LANES, 2 * LANES)] = zv
        return 0

    def zero_acc():
        lax.fori_loop(0, 256, zero_rows0, 0)
        for k in range(10):
            pltpu.sync_copy(rows0_v, acc_sp.at[pl.ds(base + k * CH, CH)])

    base = s * (R1P // NS)
    zero_acc()
    plsc.subcore_barrier()

    nblk = C0 // IB
    for r in range(2):
        for hb in range(nblk):
            pltpu.sync_copy(src_ref.at[c, r, s, pl.ds(hb * IB, IB)], src_v)
            pltpu.sync_copy(dst_ref.at[r, s, pl.ds(hb * IB, IB)], dst_v)

            for k in range(4):
                pltpu.async_copy(h_ref.at[src_v.at[k]], rows[k], gsem[k])

            def quad(qq, _):
                j = 4 * qq
                for k in range(4):
                    pltpu.make_async_copy(h_ref.at[src_v.at[j + k]], rows[k],
                                          gsem[k]).wait()
                    pltpu.async_copy(rows[k], acc_sp.at[dst_v.at[j + k]],
                                     ssem[k], add=True)

                @pl.when(qq < IB // 4 - 1)
                def _():
                    for k in range(4):
                        pltpu.make_async_copy(rows[k],
                                              acc_sp.at[dst_v.at[j + k]],
                                              ssem[k]).wait()
                        pltpu.async_copy(h_ref.at[src_v.at[j + 4 + k]],
                                         rows[k], gsem[k])
                return 0
            lax.fori_loop(0, IB // 4, quad, 0)
            for k in range(4):
                pltpu.make_async_copy(rows[k], acc_sp.at[dst_v.at[k]],
                                      ssem[k]).wait()
        plsc.subcore_barrier()

        @pl.when(c == 0)
        def _():
            pltpu.sync_copy(acc_sp.at[pl.ds(base, R1P // NS)],
                            agg_out.at[r, pl.ds(base, R1P // NS),
                                       pl.ds(0, DH)])

        @pl.when(c == 1)
        def _():
            pltpu.sync_copy(acc_sp.at[pl.ds(base, R1P // NS)],
                            agg_out.at[r, pl.ds(base, R1P // NS),
                                       pl.ds(DH, DH)])
        if r == 0:
            zero_acc()
            plsc.subcore_barrier()


_sc_layer0 = functools.partial(
    pl.kernel,
    out_type=jax.ShapeDtypeStruct((2, R1P, D), jnp.bfloat16),
    mesh=plsc.VectorSubcoreMesh(core_axis_name="c", subcore_axis_name="s"),
    compiler_params=pltpu.CompilerParams(use_tc_tiling_on_sc=False),
    scratch_types=[
        pltpu.VMEM((IB, CH), jnp.int32),
        pltpu.VMEM((IB, CH), jnp.int32),
        pltpu.VMEM((CH, DH), jnp.bfloat16),
        pltpu.VMEM((CH, DH), jnp.bfloat16),
        pltpu.VMEM((CH, DH), jnp.bfloat16),
        pltpu.VMEM((CH, DH), jnp.bfloat16),
        pltpu.VMEM_SHARED((R1P, DH), jnp.bfloat16),
    ] + [pltpu.SemaphoreType.DMA] * 8,
)(_sc_layer0_body)


def _sc_degrees_body(dst0_ref, dst1_ref, deg0_out, deg1_out,
                     dst_v, ones_v, zb16_v, deg0_sp, deg1_sp, dsem):
    c = lax.axis_index("c")
    s = lax.axis_index("s")
    zv = jnp.zeros((LANES,), jnp.float32)
    ov = jnp.ones((LANES,), jnp.float32)

    def fill16(i, _):
        ones_v[i, :] = ov
        zb16_v[i, :] = zv
        return 0
    lax.fori_loop(0, CH, fill16, 0)

    base0 = s * (R1P // NS)
    base1 = s * (R2P // NS)
    for k in range(10):
        pltpu.sync_copy(zb16_v, deg0_sp.at[pl.ds(base0 + k * CH, CH)])
    pltpu.sync_copy(zb16_v, deg1_sp.at[pl.ds(base1, CH)])
    pltpu.sync_copy(zb16_v, deg1_sp.at[pl.ds(base1 + CH, CH)])
    pltpu.sync_copy(zb16_v.at[:64], deg1_sp.at[pl.ds(base1 + 2 * CH, 64)])
    plsc.subcore_barrier()

    pltpu.sync_copy(dst0_ref.at[c, s], dst_v)

    def chunk0(j, _):
        @pl.when(j >= 8)
        def _():
            pltpu.make_async_copy(ones_v, deg0_sp.at[dst_v.at[0]],
                                  dsem).wait()
        pltpu.async_copy(ones_v, deg0_sp.at[dst_v.at[j]], dsem, add=True)
        return 0
    lax.fori_loop(0, C0, chunk0, 0)
    for _ in range(8):
        pltpu.make_async_copy(ones_v, deg0_sp.at[dst_v.at[0]], dsem).wait()

    pltpu.sync_copy(dst1_ref.at[c, s], dst_v.at[pl.ds(0, C1)])

    def chunk1(j, _):
        @pl.when(j >= 8)
        def _():
            pltpu.make_async_copy(ones_v, deg1_sp.at[dst_v.at[0]],
                                  dsem).wait()
        pltpu.async_copy(ones_v, deg1_sp.at[dst_v.at[j]], dsem, add=True)
        return 0
    lax.fori_loop(0, C1, chunk1, 0)
    for _ in range(8):
        pltpu.make_async_copy(ones_v, deg1_sp.at[dst_v.at[0]], dsem).wait()
    plsc.subcore_barrier()

    pltpu.sync_copy(deg0_sp.at[pl.ds(base0, R1P // NS)],
                    deg0_out.at[c, pl.ds(base0, R1P // NS), pl.ds(0, LANES)])
    pltpu.sync_copy(deg1_sp.at[pl.ds(base1, R2P // NS)],
                    deg1_out.at[c, pl.ds(base1, R2P // NS), pl.ds(0, LANES)])


_sc_degrees = functools.partial(
    pl.kernel,
    out_type=(jax.ShapeDtypeStruct((2, R1P, D), jnp.float32),
              jax.ShapeDtypeStruct((2, R2P, D), jnp.float32)),
    mesh=plsc.VectorSubcoreMesh(core_axis_name="c", subcore_axis_name="s"),
    compiler_params=pltpu.CompilerParams(use_tc_tiling_on_sc=False),
    scratch_types=[
        pltpu.VMEM((C0, CH), jnp.int32),
        pltpu.VMEM((CH, LANES), jnp.float32),
        pltpu.VMEM((CH, LANES), jnp.float32),
        pltpu.VMEM_SHARED((R1P, LANES), jnp.float32),
        pltpu.VMEM_SHARED((R2P, LANES), jnp.float32),
        pltpu.SemaphoreType.DMA,
    ],
)(_sc_degrees_body)


BR = 1024


def _tc_dense_body(hd_ref, agg_ref, deg_ref, ws_ref, wn_ref, b_ref, wcat_ref,
                   q_ref, p_ref):
    hd = hd_ref[...]
    a = agg_ref[...]
    d = deg_ref[...]
    deg0 = jnp.maximum(d[0, :, 0:1], 1.0)
    deg1 = jnp.maximum(d[1, :, 0:1], 1.0)
    m0 = a[0].astype(jnp.float32) / deg0
    m1 = a[1].astype(jnp.float32) / deg1
    o = (jax.nn.relu(jnp.dot(hd, ws_ref[0], preferred_element_type=jnp.float32)
                     + jnp.dot(m0, wn_ref[0], preferred_element_type=jnp.float32)
                     + b_ref[0])
         + jax.nn.relu(jnp.dot(hd, ws_ref[1], preferred_element_type=jnp.float32)
                       + jnp.dot(m1, wn_ref[1], preferred_element_type=jnp.float32)
                       + b_ref[1]))
    qp = jnp.dot(o, wcat_ref[...], preferred_element_type=jnp.float32)
    q_ref[0] = jnp.broadcast_to(qp[:, 0:1], (BR, LANES))
    q_ref[1] = jnp.broadcast_to(qp[:, 1:2], (BR, LANES))
    p_ref[...] = jnp.broadcast_to(qp[:, 2:3], (BR, LANES))


def _tc_dense(hd, agg, deg, ws, wn, b2, wcat):
    nb = R1P // BR
    return pl.pallas_call(
        _tc_dense_body,
        grid=(nb,),
        in_specs=[
            pl.BlockSpec((BR, D), lambda i: (i, 0)),
            pl.BlockSpec((2, BR, D), lambda i: (0, i, 0)),
            pl.BlockSpec((2, BR, D), lambda i: (0, i, 0)),
            pl.BlockSpec((2, D, D), lambda i: (0, 0, 0)),
            pl.BlockSpec((2, D, D), lambda i: (0, 0, 0)),
            pl.BlockSpec((2, 1, D), lambda i: (0, 0, 0)),
            pl.BlockSpec((D, D), lambda i: (0, 0)),
        ],
        out_specs=[
            pl.BlockSpec((2, BR, LANES), lambda i: (0, i, 0)),
            pl.BlockSpec((BR, LANES), lambda i: (i, 0)),
        ],
        out_shape=[
            jax.ShapeDtypeStruct((2, R1P, LANES), jnp.float32),
            jax.ShapeDtypeStruct((R1P, LANES), jnp.float32),
        ],
    )(hd, agg, deg, ws, wn, b2, wcat)


def _sc_layer1_body(q_ref, src_ref, dst_ref, sd_out,
                    src_v, dst_v, rows0_v, rows1_v, zb16_v, s_sp,
                    gsem0, gsem1, ssem0, ssem1):
    c = lax.axis_index("c")
    s = lax.axis_index("s")
    zv = jnp.zeros((LANES,), jnp.float32)

    def fill16(i, _):
        zb16_v[i, :] = zv
        return 0
    lax.fori_loop(0, CH, fill16, 0)

    base = s * (R2P // NS)
    pltpu.sync_copy(zb16_v, s_sp.at[pl.ds(base, CH)])
    pltpu.sync_copy(zb16_v, s_sp.at[pl.ds(base + CH, CH)])
    pltpu.sync_copy(zb16_v.at[:64], s_sp.at[pl.ds(base + 2 * CH, 64)])
    plsc.subcore_barrier()

    pltpu.sync_copy(src_ref.at[c, s], src_v)
    pltpu.sync_copy(dst_ref.at[c, s], dst_v)

    pltpu.async_copy(q_ref.at[src_v.at[0]], rows0_v, gsem0)
    pltpu.async_copy(q_ref.at[src_v.at[1]], rows1_v, gsem1)

    def pair(jj, _):
        j0 = 2 * jj
        pltpu.make_async_copy(q_ref.at[src_v.at[j0]], rows0_v, gsem0).wait()
        pltpu.async_copy(rows0_v, s_sp.at[dst_v.at[j0]], ssem0, add=True)
        pltpu.make_async_copy(q_ref.at[src_v.at[j0 + 1]], rows1_v,
                              gsem1).wait()
        pltpu.async_copy(rows1_v, s_sp.at[dst_v.at[j0 + 1]], ssem1, add=True)

        @pl.when(jj < C1 // 2 - 1)
        def _():
            pltpu.make_async_copy(rows0_v, s_sp.at[dst_v.at[j0]],
                                  ssem0).wait()
            pltpu.async_copy(q_ref.at[src_v.at[j0 + 2]], rows0_v, gsem0)
            pltpu.make_async_copy(rows1_v, s_sp.at[dst_v.at[j0 + 1]],
                                  ssem1).wait()
            pltpu.async_copy(q_ref.at[src_v.at[j0 + 3]], rows1_v, gsem1)
        return 0
    lax.fori_loop(0, C1 // 2, pair, 0)
    pltpu.make_async_copy(rows0_v, s_sp.at[dst_v.at[0]], ssem0).wait()
    pltpu.make_async_copy(rows1_v, s_sp.at[dst_v.at[1]], ssem1).wait()
    plsc.subcore_barrier()

    nrow = R2P // NS

    @pl.when(c == 0)
    def _():
        pltpu.sync_copy(s_sp.at[pl.ds(base, nrow)],
                        sd_out.at[pl.ds(base, nrow), pl.ds(0, LANES)])

    @pl.when(c == 1)
    def _():
        pltpu.sync_copy(s_sp.at[pl.ds(base, nrow)],
                        sd_out.at[pl.ds(base, nrow), pl.ds(LANES, LANES)])


_sc_layer1 = functools.partial(
    pl.kernel,
    out_type=jax.ShapeDtypeStruct((R2P, D), jnp.float32),
    mesh=plsc.VectorSubcoreMesh(core_axis_name="c", subcore_axis_name="s"),
    compiler_params=pltpu.CompilerParams(use_tc_tiling_on_sc=False),
    scratch_types=[
        pltpu.VMEM((C1, CH), jnp.int32),
        pltpu.VMEM((C1, CH), jnp.int32),
        pltpu.VMEM((CH, LANES), jnp.float32),
        pltpu.VMEM((CH, LANES), jnp.float32),
        pltpu.VMEM((CH, LANES), jnp.float32),
        pltpu.VMEM_SHARED((R2P, LANES), jnp.float32),
        pltpu.SemaphoreType.DMA,
        pltpu.SemaphoreType.DMA,
        pltpu.SemaphoreType.DMA,
        pltpu.SemaphoreType.DMA,
    ],
)(_sc_layer1_body)


def _tc_final_body(p_ref, sd_ref, dg_ref, b_ref, o_ref):
    p = p_ref[:, 0:1]
    sd = sd_ref[...]
    dg = dg_ref[...]
    s0 = sd[:, 0:1] / jnp.maximum(dg[0, :, 0:1], 1.0)
    s1 = sd[:, LANES:LANES + 1] / jnp.maximum(dg[1, :, 0:1], 1.0)
    o_ref[...] = jnp.broadcast_to(
        jax.nn.sigmoid(p + s0 + s1 + b_ref[0, 0:1]), (R2P, LANES))


def _tc_final(p, sd, deg1, bsum):
    return pl.pallas_call(
        _tc_final_body,
        grid=(1,),
        in_specs=[
            pl.BlockSpec((R2P, LANES), lambda i: (0, 0)),
            pl.BlockSpec((R2P, D), lambda i: (0, 0)),
            pl.BlockSpec((2, R2P, D), lambda i: (0, 0, 0)),
            pl.BlockSpec((1, LANES), lambda i: (0, 0)),
        ],
        out_specs=pl.BlockSpec((R2P, LANES), lambda i: (0, 0)),
        out_shape=jax.ShapeDtypeStruct((R2P, LANES), jnp.float32),
    )(p, sd, deg1, bsum)


def kernel(h, src0_r0, dst0_r0, src0_r1, dst0_r1, src1_r0, dst1_r0,
           src1_r1, dst1_r1, Wself0_r0, Wneigh0_r0, b0_r0, Wself0_r1,
           Wneigh0_r1, b0_r1, Wself1_r0, Wneigh1_r0, b1_r0, Wself1_r1,
           Wneigh1_r1, b1_r1):
    hflat = h.astype(jnp.bfloat16).reshape(2 * N0, DH)

    s0p, d0p = _pad_edges(src0_r0, dst0_r0, EP0, N0, N1, R1P)
    s1p, d1p = _pad_edges(src0_r1, dst0_r1, EP0, N0, N1, R1P)
    srcA = jnp.stack([
        jnp.stack([(2 * s0p).reshape(NS, C0, CH),
                   (2 * s1p).reshape(NS, C0, CH)]),
        jnp.stack([(2 * s0p + 1).reshape(NS, C0, CH),
                   (2 * s1p + 1).reshape(NS, C0, CH)]),
    ])
    dstA = jnp.stack([d0p.reshape(NS, C0, CH), d1p.reshape(NS, C0, CH)])

    agg = _sc_layer0(hflat, srcA, dstA)

    ws = jnp.stack([Wself0_r0, Wself0_r1])
    wn = jnp.stack([Wneigh0_r0, Wneigh0_r1])
    b2 = jnp.stack([b0_r0, b0_r1]).reshape(2, 1, D)
    wcat = jnp.concatenate(
        [Wneigh1_r0, Wneigh1_r1, Wself1_r0 + Wself1_r1,
         jnp.zeros((D, D - 3), jnp.float32)], axis=1)

    sb0, db0 = _pad_edges(src1_r0, dst1_r0, EP1, N1, N2, R2P)
    sb1, db1 = _pad_edges(src1_r1, dst1_r1, EP1, N1, N2, R2P)
    srcB = jnp.stack([sb0.reshape(NS, C1, CH),
                      (sb1 + R1P).reshape(NS, C1, CH)])
    dstB = jnp.stack([db0.reshape(NS, C1, CH), db1.reshape(NS, C1, CH)])

    deg0, deg1 = _sc_degrees(dstA, dstB)

    q2, p = _tc_dense(h, agg, deg0, ws, wn, b2, wcat)
    qf = q2.reshape(2 * R1P, LANES)

    sd = _sc_layer1(qf, srcB, dstB)

    bsum = jnp.broadcast_to((b1_r0 + b1_r1).reshape(1, 1), (1, LANES))
    out = _tc_final(p, sd, deg1, bsum)
    return out[:N2, 0:1]


# speedup vs baseline: 1.1569x; 1.0209x over previous
import functools

import jax
import jax.numpy as jnp
from jax import lax
from jax.experimental import pallas as pl
from jax.experimental.pallas import tpu as pltpu
from jax.experimental.pallas import tpu_sc as plsc

N0, N1, N2 = 50000, 20000, 5000
E0, E1 = 320000, 80000
D, DH = 128, 64

NC, NS, LANES = 2, 16, 16
CH = 128

C0 = 160
IB = 40
T0E = C0 * CH
EP0 = NS * T0E
R1P = 20480

C1 = 40
T1E = C1 * CH
EP1 = NS * T1E
R2P = 5120


def _pad_edges(src, dst, ep, n_src, n_dst, n_dst_pad):
    pe = ep - src.shape[0]
    ar = jnp.arange(pe, dtype=jnp.int32)
    src_p = jnp.concatenate([src, ar % n_src])
    dst_p = jnp.concatenate([dst, n_dst + ar % (n_dst_pad - n_dst)])
    return src_p, dst_p


def _sc_layer0_body(h_ref, src_ref, dst_ref, agg_out,
                    src_v, dst_v, rows0_v, rows1_v, rows2_v, rows3_v, acc_sp,
                    g0, g1, g2, g3, s0, s1, s2, s3):
    c = lax.axis_index("c")
    s = lax.axis_index("s")
    zv = jnp.zeros((2 * LANES,), jnp.bfloat16)
    rows = (rows0_v, rows1_v, rows2_v, rows3_v)
    gsem = (g0, g1, g2, g3)
    ssem = (s0, s1, s2, s3)

    def zero_rows0(i, _):
        r, q = i // 2, i % 2
        rows0_v[r, pl.ds(q * 2 * LANES, 2 * LANES)] = zv
        return 0

    def zero_acc():
        lax.fori_loop(0, 256, zero_rows0, 0)
        for k in range(10):
            pltpu.sync_copy(rows0_v, acc_sp.at[pl.ds(base + k * CH, CH)])

    base = s * (R1P // NS)
    zero_acc()
    plsc.subcore_barrier()

    nblk = C0 // IB
    for r in range(2):
        for hb in range(nblk):
            pltpu.sync_copy(src_ref.at[c, r, s, pl.ds(hb * IB, IB)], src_v)
            pltpu.sync_copy(dst_ref.at[r, s, pl.ds(hb * IB, IB)], dst_v)

            for k in range(4):
                pltpu.async_copy(h_ref.at[src_v.at[k]], rows[k], gsem[k])

            def quad(qq, _):
                j = 4 * qq
                for k in range(4):
                    pltpu.make_async_copy(h_ref.at[src_v.at[j + k]], rows[k],
                                          gsem[k]).wait()
                    pltpu.async_copy(rows[k], acc_sp.at[dst_v.at[j + k]],
                                     ssem[k], add=True)

                @pl.when(qq < IB // 4 - 1)
                def _():
                    for k in range(4):
                        pltpu.make_async_copy(rows[k],
                                              acc_sp.at[dst_v.at[j + k]],
                                              ssem[k]).wait()
                        pltpu.async_copy(h_ref.at[src_v.at[j + 4 + k]],
                                         rows[k], gsem[k])
                return 0
            lax.fori_loop(0, IB // 4, quad, 0)
            for k in range(4):
                pltpu.make_async_copy(rows[k], acc_sp.at[dst_v.at[k]],
                                      ssem[k]).wait()
        plsc.subcore_barrier()

        @pl.when(c == 0)
        def _():
            pltpu.sync_copy(acc_sp.at[pl.ds(base, R1P // NS)],
                            agg_out.at[r, pl.ds(base, R1P // NS),
                                       pl.ds(0, DH)])

        @pl.when(c == 1)
        def _():
            pltpu.sync_copy(acc_sp.at[pl.ds(base, R1P // NS)],
                            agg_out.at[r, pl.ds(base, R1P // NS),
                                       pl.ds(DH, DH)])
        if r == 0:
            zero_acc()
            plsc.subcore_barrier()


_sc_layer0 = functools.partial(
    pl.kernel,
    out_type=jax.ShapeDtypeStruct((2, R1P, D), jnp.bfloat16),
    mesh=plsc.VectorSubcoreMesh(core_axis_name="c", subcore_axis_name="s"),
    compiler_params=pltpu.CompilerParams(use_tc_tiling_on_sc=False),
    scratch_types=[
        pltpu.VMEM((IB, CH), jnp.int32),
        pltpu.VMEM((IB, CH), jnp.int32),
        pltpu.VMEM((CH, DH), jnp.bfloat16),
        pltpu.VMEM((CH, DH), jnp.bfloat16),
        pltpu.VMEM((CH, DH), jnp.bfloat16),
        pltpu.VMEM((CH, DH), jnp.bfloat16),
        pltpu.VMEM_SHARED((R1P, DH), jnp.bfloat16),
    ] + [pltpu.SemaphoreType.DMA] * 8,
)(_sc_layer0_body)


def _sc_degrees_body(dst0_ref, dst1_ref, deg0_out, deg1_out,
                     dst_v, ones_v, zb16_v, deg0_sp, deg1_sp, dsem):
    c = lax.axis_index("c")
    s = lax.axis_index("s")
    zv = jnp.zeros((LANES,), jnp.float32)
    ov = jnp.ones((LANES,), jnp.float32)

    def fill16(i, _):
        ones_v[i, :] = ov
        zb16_v[i, :] = zv
        return 0
    lax.fori_loop(0, CH, fill16, 0)

    base0 = s * (R1P // NS)
    base1 = s * (R2P // NS)
    for k in range(10):
        pltpu.sync_copy(zb16_v, deg0_sp.at[pl.ds(base0 + k * CH, CH)])
    pltpu.sync_copy(zb16_v, deg1_sp.at[pl.ds(base1, CH)])
    pltpu.sync_copy(zb16_v, deg1_sp.at[pl.ds(base1 + CH, CH)])
    pltpu.sync_copy(zb16_v.at[:64], deg1_sp.at[pl.ds(base1 + 2 * CH, 64)])
    plsc.subcore_barrier()

    pltpu.sync_copy(dst0_ref.at[c, s], dst_v)

    def chunk0(j, _):
        @pl.when(j >= 8)
        def _():
            pltpu.make_async_copy(ones_v, deg0_sp.at[dst_v.at[0]],
                                  dsem).wait()
        pltpu.async_copy(ones_v, deg0_sp.at[dst_v.at[j]], dsem, add=True)
        return 0
    lax.fori_loop(0, C0, chunk0, 0)
    for _ in range(8):
        pltpu.make_async_copy(ones_v, deg0_sp.at[dst_v.at[0]], dsem).wait()

    pltpu.sync_copy(dst1_ref.at[c, s], dst_v.at[pl.ds(0, C1)])

    def chunk1(j, _):
        @pl.when(j >= 8)
        def _():
            pltpu.make_async_copy(ones_v, deg1_sp.at[dst_v.at[0]],
                                  dsem).wait()
        pltpu.async_copy(ones_v, deg1_sp.at[dst_v.at[j]], dsem, add=True)
        return 0
    lax.fori_loop(0, C1, chunk1, 0)
    for _ in range(8):
        pltpu.make_async_copy(ones_v, deg1_sp.at[dst_v.at[0]], dsem).wait()
    plsc.subcore_barrier()

    pltpu.sync_copy(deg0_sp.at[pl.ds(base0, R1P // NS)],
                    deg0_out.at[c, pl.ds(base0, R1P // NS), pl.ds(0, LANES)])
    pltpu.sync_copy(deg1_sp.at[pl.ds(base1, R2P // NS)],
                    deg1_out.at[c, pl.ds(base1, R2P // NS), pl.ds(0, LANES)])


_sc_degrees = functools.partial(
    pl.kernel,
    out_type=(jax.ShapeDtypeStruct((2, R1P, D), jnp.float32),
              jax.ShapeDtypeStruct((2, R2P, D), jnp.float32)),
    mesh=plsc.VectorSubcoreMesh(core_axis_name="c", subcore_axis_name="s"),
    compiler_params=pltpu.CompilerParams(use_tc_tiling_on_sc=False),
    scratch_types=[
        pltpu.VMEM((C0, CH), jnp.int32),
        pltpu.VMEM((CH, LANES), jnp.float32),
        pltpu.VMEM((CH, LANES), jnp.float32),
        pltpu.VMEM_SHARED((R1P, LANES), jnp.float32),
        pltpu.VMEM_SHARED((R2P, LANES), jnp.float32),
        pltpu.SemaphoreType.DMA,
    ],
)(_sc_degrees_body)


BR = 2048


def _tc_dense_body(hd_ref, agg_ref, deg_ref, ws_ref, wn_ref, b_ref, wcat_ref,
                   q_ref, p_ref):
    hd = hd_ref[...]
    a = agg_ref[...]
    d = deg_ref[...]
    deg0 = jnp.maximum(d[0, :, 0:1], 1.0)
    deg1 = jnp.maximum(d[1, :, 0:1], 1.0)
    m0 = a[0].astype(jnp.float32) / deg0
    m1 = a[1].astype(jnp.float32) / deg1
    o = (jax.nn.relu(jnp.dot(hd, ws_ref[0], preferred_element_type=jnp.float32)
                     + jnp.dot(m0, wn_ref[0], preferred_element_type=jnp.float32)
                     + b_ref[0])
         + jax.nn.relu(jnp.dot(hd, ws_ref[1], preferred_element_type=jnp.float32)
                       + jnp.dot(m1, wn_ref[1], preferred_element_type=jnp.float32)
                       + b_ref[1]))
    qp = jnp.dot(o, wcat_ref[...], preferred_element_type=jnp.float32)
    q_ref[0] = jnp.broadcast_to(qp[:, 0:1], (BR, LANES))
    q_ref[1] = jnp.broadcast_to(qp[:, 1:2], (BR, LANES))
    p_ref[...] = jnp.broadcast_to(qp[:, 2:3], (BR, LANES))


def _tc_dense(hd, agg, deg, ws, wn, b2, wcat):
    nb = R1P // BR
    return pl.pallas_call(
        _tc_dense_body,
        grid=(nb,),
        in_specs=[
            pl.BlockSpec((BR, D), lambda i: (i, 0)),
            pl.BlockSpec((2, BR, D), lambda i: (0, i, 0)),
            pl.BlockSpec((2, BR, D), lambda i: (0, i, 0)),
            pl.BlockSpec((2, D, D), lambda i: (0, 0, 0)),
            pl.BlockSpec((2, D, D), lambda i: (0, 0, 0)),
            pl.BlockSpec((2, 1, D), lambda i: (0, 0, 0)),
            pl.BlockSpec((D, D), lambda i: (0, 0)),
        ],
        out_specs=[
            pl.BlockSpec((2, BR, LANES), lambda i: (0, i, 0)),
            pl.BlockSpec((BR, LANES), lambda i: (i, 0)),
        ],
        out_shape=[
            jax.ShapeDtypeStruct((2, R1P, LANES), jnp.float32),
            jax.ShapeDtypeStruct((R1P, LANES), jnp.float32),
        ],
    )(hd, agg, deg, ws, wn, b2, wcat)


def _sc_layer1_body(q_ref, src_ref, dst_ref, sd_out,
                    src_v, dst_v, rows0_v, rows1_v, zb16_v, s_sp,
                    gsem0, gsem1, ssem0, ssem1):
    c = lax.axis_index("c")
    s = lax.axis_index("s")
    zv = jnp.zeros((LANES,), jnp.float32)

    def fill16(i, _):
        zb16_v[i, :] = zv
        return 0
    lax.fori_loop(0, CH, fill16, 0)

    base = s * (R2P // NS)
    pltpu.sync_copy(zb16_v, s_sp.at[pl.ds(base, CH)])
    pltpu.sync_copy(zb16_v, s_sp.at[pl.ds(base + CH, CH)])
    pltpu.sync_copy(zb16_v.at[:64], s_sp.at[pl.ds(base + 2 * CH, 64)])
    plsc.subcore_barrier()

    pltpu.sync_copy(src_ref.at[c, s], src_v)
    pltpu.sync_copy(dst_ref.at[c, s], dst_v)

    pltpu.async_copy(q_ref.at[src_v.at[0]], rows0_v, gsem0)
    pltpu.async_copy(q_ref.at[src_v.at[1]], rows1_v, gsem1)

    def pair(jj, _):
        j0 = 2 * jj
        pltpu.make_async_copy(q_ref.at[src_v.at[j0]], rows0_v, gsem0).wait()
        pltpu.async_copy(rows0_v, s_sp.at[dst_v.at[j0]], ssem0, add=True)
        pltpu.make_async_copy(q_ref.at[src_v.at[j0 + 1]], rows1_v,
                              gsem1).wait()
        pltpu.async_copy(rows1_v, s_sp.at[dst_v.at[j0 + 1]], ssem1, add=True)

        @pl.when(jj < C1 // 2 - 1)
        def _():
            pltpu.make_async_copy(rows0_v, s_sp.at[dst_v.at[j0]],
                                  ssem0).wait()
            pltpu.async_copy(q_ref.at[src_v.at[j0 + 2]], rows0_v, gsem0)
            pltpu.make_async_copy(rows1_v, s_sp.at[dst_v.at[j0 + 1]],
                                  ssem1).wait()
            pltpu.async_copy(q_ref.at[src_v.at[j0 + 3]], rows1_v, gsem1)
        return 0
    lax.fori_loop(0, C1 // 2, pair, 0)
    pltpu.make_async_copy(rows0_v, s_sp.at[dst_v.at[0]], ssem0).wait()
    pltpu.make_async_copy(rows1_v, s_sp.at[dst_v.at[1]], ssem1).wait()
    plsc.subcore_barrier()

    nrow = R2P // NS

    @pl.when(c == 0)
    def _():
        pltpu.sync_copy(s_sp.at[pl.ds(base, nrow)],
                        sd_out.at[pl.ds(base, nrow), pl.ds(0, LANES)])

    @pl.when(c == 1)
    def _():
        pltpu.sync_copy(s_sp.at[pl.ds(base, nrow)],
                        sd_out.at[pl.ds(base, nrow), pl.ds(LANES, LANES)])


_sc_layer1 = functools.partial(
    pl.kernel,
    out_type=jax.ShapeDtypeStruct((R2P, D), jnp.float32),
    mesh=plsc.VectorSubcoreMesh(core_axis_name="c", subcore_axis_name="s"),
    compiler_params=pltpu.CompilerParams(use_tc_tiling_on_sc=False),
    scratch_types=[
        pltpu.VMEM((C1, CH), jnp.int32),
        pltpu.VMEM((C1, CH), jnp.int32),
        pltpu.VMEM((CH, LANES), jnp.float32),
        pltpu.VMEM((CH, LANES), jnp.float32),
        pltpu.VMEM((CH, LANES), jnp.float32),
        pltpu.VMEM_SHARED((R2P, LANES), jnp.float32),
        pltpu.SemaphoreType.DMA,
        pltpu.SemaphoreType.DMA,
        pltpu.SemaphoreType.DMA,
        pltpu.SemaphoreType.DMA,
    ],
)(_sc_layer1_body)


def _tc_final_body(p_ref, sd_ref, dg_ref, b_ref, o_ref):
    p = p_ref[:, 0:1]
    sd = sd_ref[...]
    dg = dg_ref[...]
    s0 = sd[:, 0:1] / jnp.maximum(dg[0, :, 0:1], 1.0)
    s1 = sd[:, LANES:LANES + 1] / jnp.maximum(dg[1, :, 0:1], 1.0)
    o_ref[...] = jnp.broadcast_to(
        jax.nn.sigmoid(p + s0 + s1 + b_ref[0, 0:1]), (R2P, LANES))


def _tc_final(p, sd, deg1, bsum):
    return pl.pallas_call(
        _tc_final_body,
        grid=(1,),
        in_specs=[
            pl.BlockSpec((R2P, LANES), lambda i: (0, 0)),
            pl.BlockSpec((R2P, D), lambda i: (0, 0)),
            pl.BlockSpec((2, R2P, D), lambda i: (0, 0, 0)),
            pl.BlockSpec((1, LANES), lambda i: (0, 0)),
        ],
        out_specs=pl.BlockSpec((R2P, LANES), lambda i: (0, 0)),
        out_shape=jax.ShapeDtypeStruct((R2P, LANES), jnp.float32),
    )(p, sd, deg1, bsum)


def kernel(h, src0_r0, dst0_r0, src0_r1, dst0_r1, src1_r0, dst1_r0,
           src1_r1, dst1_r1, Wself0_r0, Wneigh0_r0, b0_r0, Wself0_r1,
           Wneigh0_r1, b0_r1, Wself1_r0, Wneigh1_r0, b1_r0, Wself1_r1,
           Wneigh1_r1, b1_r1):
    hflat = h.astype(jnp.bfloat16).reshape(2 * N0, DH)

    s0p, d0p = _pad_edges(src0_r0, dst0_r0, EP0, N0, N1, R1P)
    s1p, d1p = _pad_edges(src0_r1, dst0_r1, EP0, N0, N1, R1P)
    srcA = jnp.stack([
        jnp.stack([(2 * s0p).reshape(NS, C0, CH),
                   (2 * s1p).reshape(NS, C0, CH)]),
        jnp.stack([(2 * s0p + 1).reshape(NS, C0, CH),
                   (2 * s1p + 1).reshape(NS, C0, CH)]),
    ])
    dstA = jnp.stack([d0p.reshape(NS, C0, CH), d1p.reshape(NS, C0, CH)])

    agg = _sc_layer0(hflat, srcA, dstA)

    ws = jnp.stack([Wself0_r0, Wself0_r1])
    wn = jnp.stack([Wneigh0_r0, Wneigh0_r1])
    b2 = jnp.stack([b0_r0, b0_r1]).reshape(2, 1, D)
    wcat = jnp.concatenate(
        [Wneigh1_r0, Wneigh1_r1, Wself1_r0 + Wself1_r1,
         jnp.zeros((D, D - 3), jnp.float32)], axis=1)

    sb0, db0 = _pad_edges(src1_r0, dst1_r0, EP1, N1, N2, R2P)
    sb1, db1 = _pad_edges(src1_r1, dst1_r1, EP1, N1, N2, R2P)
    srcB = jnp.stack([sb0.reshape(NS, C1, CH),
                      (sb1 + R1P).reshape(NS, C1, CH)])
    dstB = jnp.stack([db0.reshape(NS, C1, CH), db1.reshape(NS, C1, CH)])

    deg0, deg1 = _sc_degrees(dstA, dstB)

    q2, p = _tc_dense(h, agg, deg0, ws, wn, b2, wcat)
    qf = q2.reshape(2 * R1P, LANES)

    sd = _sc_layer1(qf, srcB, dstB)

    bsum = jnp.broadcast_to((b1_r0 + b1_r1).reshape(1, 1), (1, LANES))
    out = _tc_final(p, sd, deg1, bsum)
    return out[:N2, 0:1]


# speedup vs baseline: 1.1602x; 1.0029x over previous
import functools

import jax
import jax.numpy as jnp
from jax import lax
from jax.experimental import pallas as pl
from jax.experimental.pallas import tpu as pltpu
from jax.experimental.pallas import tpu_sc as plsc

N0, N1, N2 = 50000, 20000, 5000
E0, E1 = 320000, 80000
D, DH = 128, 64

NC, NS, LANES = 2, 16, 16
CH = 128

C0 = 160
IB = 40
T0E = C0 * CH
EP0 = NS * T0E
R1P = 20480

C1 = 40
T1E = C1 * CH
EP1 = NS * T1E
R2P = 5120


def _pad_edges(src, dst, ep, n_src, n_dst, n_dst_pad):
    pe = ep - src.shape[0]
    ar = jnp.arange(pe, dtype=jnp.int32)
    src_p = jnp.concatenate([src, ar % n_src])
    dst_p = jnp.concatenate([dst, n_dst + ar % (n_dst_pad - n_dst)])
    return src_p, dst_p


def _sc_layer0_body(h_ref, src_ref, dst_ref, agg_out,
                    src_v, dst_v, rows0_v, rows1_v, rows2_v, rows3_v, acc_sp,
                    g0, g1, g2, g3, s0, s1, s2, s3):
    c = lax.axis_index("c")
    s = lax.axis_index("s")
    zv = jnp.zeros((2 * LANES,), jnp.bfloat16)
    rows = (rows0_v, rows1_v, rows2_v, rows3_v)
    gsem = (g0, g1, g2, g3)
    ssem = (s0, s1, s2, s3)

    def zero_rows0(i, _):
        r, q = i // 2, i % 2
        rows0_v[r, pl.ds(q * 2 * LANES, 2 * LANES)] = zv
        return 0

    def zero_acc():
        lax.fori_loop(0, 256, zero_rows0, 0)
        for k in range(10):
            pltpu.sync_copy(rows0_v, acc_sp.at[pl.ds(base + k * CH, CH)])

    base = s * (R1P // NS)
    zero_acc()
    plsc.subcore_barrier()

    nblk = C0 // IB
    for r in range(2):
        for hb in range(nblk):
            pltpu.sync_copy(src_ref.at[c, r, s, pl.ds(hb * IB, IB)], src_v)
            pltpu.sync_copy(dst_ref.at[r, s, pl.ds(hb * IB, IB)], dst_v)

            for k in range(4):
                pltpu.async_copy(h_ref.at[src_v.at[k]], rows[k], gsem[k])

            def quad(qq, _):
                j = 4 * qq
                for k in range(4):
                    pltpu.make_async_copy(h_ref.at[src_v.at[j + k]], rows[k],
                                          gsem[k]).wait()
                    pltpu.async_copy(rows[k], acc_sp.at[dst_v.at[j + k]],
                                     ssem[k], add=True)

                @pl.when(qq < IB // 4 - 1)
                def _():
                    for k in range(4):
                        pltpu.make_async_copy(rows[k],
                                              acc_sp.at[dst_v.at[j + k]],
                                              ssem[k]).wait()
                        pltpu.async_copy(h_ref.at[src_v.at[j + 4 + k]],
                                         rows[k], gsem[k])
                return 0
            lax.fori_loop(0, IB // 4, quad, 0)
            for k in range(4):
                pltpu.make_async_copy(rows[k], acc_sp.at[dst_v.at[k]],
                                      ssem[k]).wait()
        plsc.subcore_barrier()

        @pl.when(c == 0)
        def _():
            pltpu.sync_copy(acc_sp.at[pl.ds(base, R1P // NS)],
                            agg_out.at[r, pl.ds(base, R1P // NS),
                                       pl.ds(0, DH)])

        @pl.when(c == 1)
        def _():
            pltpu.sync_copy(acc_sp.at[pl.ds(base, R1P // NS)],
                            agg_out.at[r, pl.ds(base, R1P // NS),
                                       pl.ds(DH, DH)])
        if r == 0:
            zero_acc()
            plsc.subcore_barrier()


_sc_layer0 = functools.partial(
    pl.kernel,
    out_type=jax.ShapeDtypeStruct((2, R1P, D), jnp.bfloat16),
    mesh=plsc.VectorSubcoreMesh(core_axis_name="c", subcore_axis_name="s"),
    compiler_params=pltpu.CompilerParams(use_tc_tiling_on_sc=False),
    scratch_types=[
        pltpu.VMEM((IB, CH), jnp.int32),
        pltpu.VMEM((IB, CH), jnp.int32),
        pltpu.VMEM((CH, DH), jnp.bfloat16),
        pltpu.VMEM((CH, DH), jnp.bfloat16),
        pltpu.VMEM((CH, DH), jnp.bfloat16),
        pltpu.VMEM((CH, DH), jnp.bfloat16),
        pltpu.VMEM_SHARED((R1P, DH), jnp.bfloat16),
    ] + [pltpu.SemaphoreType.DMA] * 8,
)(_sc_layer0_body)


def _sc_degrees_body(dst0_ref, dst1_ref, deg0_out, deg1_out,
                     dst_v, ones_v, zb16_v, deg0_sp, deg1_sp, dsem):
    c = lax.axis_index("c")
    s = lax.axis_index("s")
    zv = jnp.zeros((LANES,), jnp.float32)
    ov = jnp.ones((LANES,), jnp.float32)

    def fill16(i, _):
        ones_v[i, :] = ov
        zb16_v[i, :] = zv
        return 0
    lax.fori_loop(0, CH, fill16, 0)

    base0 = s * (R1P // NS)
    base1 = s * (R2P // NS)
    for k in range(10):
        pltpu.sync_copy(zb16_v, deg0_sp.at[pl.ds(base0 + k * CH, CH)])
    pltpu.sync_copy(zb16_v, deg1_sp.at[pl.ds(base1, CH)])
    pltpu.sync_copy(zb16_v, deg1_sp.at[pl.ds(base1 + CH, CH)])
    pltpu.sync_copy(zb16_v.at[:64], deg1_sp.at[pl.ds(base1 + 2 * CH, 64)])
    plsc.subcore_barrier()

    pltpu.sync_copy(dst0_ref.at[c, s], dst_v)

    def chunk0(j, _):
        @pl.when(j >= 8)
        def _():
            pltpu.make_async_copy(ones_v, deg0_sp.at[dst_v.at[0]],
                                  dsem).wait()
        pltpu.async_copy(ones_v, deg0_sp.at[dst_v.at[j]], dsem, add=True)
        return 0
    lax.fori_loop(0, C0, chunk0, 0)
    for _ in range(8):
        pltpu.make_async_copy(ones_v, deg0_sp.at[dst_v.at[0]], dsem).wait()

    pltpu.sync_copy(dst1_ref.at[c, s], dst_v.at[pl.ds(0, C1)])

    def chunk1(j, _):
        @pl.when(j >= 8)
        def _():
            pltpu.make_async_copy(ones_v, deg1_sp.at[dst_v.at[0]],
                                  dsem).wait()
        pltpu.async_copy(ones_v, deg1_sp.at[dst_v.at[j]], dsem, add=True)
        return 0
    lax.fori_loop(0, C1, chunk1, 0)
    for _ in range(8):
        pltpu.make_async_copy(ones_v, deg1_sp.at[dst_v.at[0]], dsem).wait()
    plsc.subcore_barrier()

    pltpu.sync_copy(deg0_sp.at[pl.ds(base0, R1P // NS)],
                    deg0_out.at[c, pl.ds(base0, R1P // NS), pl.ds(0, LANES)])
    pltpu.sync_copy(deg1_sp.at[pl.ds(base1, R2P // NS)],
                    deg1_out.at[c, pl.ds(base1, R2P // NS), pl.ds(0, LANES)])


_sc_degrees = functools.partial(
    pl.kernel,
    out_type=(jax.ShapeDtypeStruct((2, R1P, D), jnp.float32),
              jax.ShapeDtypeStruct((2, R2P, D), jnp.float32)),
    mesh=plsc.VectorSubcoreMesh(core_axis_name="c", subcore_axis_name="s"),
    compiler_params=pltpu.CompilerParams(use_tc_tiling_on_sc=False),
    scratch_types=[
        pltpu.VMEM((C0, CH), jnp.int32),
        pltpu.VMEM((CH, LANES), jnp.float32),
        pltpu.VMEM((CH, LANES), jnp.float32),
        pltpu.VMEM_SHARED((R1P, LANES), jnp.float32),
        pltpu.VMEM_SHARED((R2P, LANES), jnp.float32),
        pltpu.SemaphoreType.DMA,
    ],
)(_sc_degrees_body)


BR = 4096


def _tc_dense_body(hd_ref, agg_ref, deg_ref, ws_ref, wn_ref, b_ref, wcat_ref,
                   q_ref, p_ref):
    hd = hd_ref[...]
    a = agg_ref[...]
    d = deg_ref[...]
    deg0 = jnp.maximum(d[0, :, 0:1], 1.0)
    deg1 = jnp.maximum(d[1, :, 0:1], 1.0)
    m0 = a[0].astype(jnp.float32) / deg0
    m1 = a[1].astype(jnp.float32) / deg1
    o = (jax.nn.relu(jnp.dot(hd, ws_ref[0], preferred_element_type=jnp.float32)
                     + jnp.dot(m0, wn_ref[0], preferred_element_type=jnp.float32)
                     + b_ref[0])
         + jax.nn.relu(jnp.dot(hd, ws_ref[1], preferred_element_type=jnp.float32)
                       + jnp.dot(m1, wn_ref[1], preferred_element_type=jnp.float32)
                       + b_ref[1]))
    qp = jnp.dot(o, wcat_ref[...], preferred_element_type=jnp.float32)
    q_ref[0] = jnp.broadcast_to(qp[:, 0:1], (BR, LANES))
    q_ref[1] = jnp.broadcast_to(qp[:, 1:2], (BR, LANES))
    p_ref[...] = jnp.broadcast_to(qp[:, 2:3], (BR, LANES))


def _tc_dense(hd, agg, deg, ws, wn, b2, wcat):
    nb = R1P // BR
    return pl.pallas_call(
        _tc_dense_body,
        grid=(nb,),
        in_specs=[
            pl.BlockSpec((BR, D), lambda i: (i, 0)),
            pl.BlockSpec((2, BR, D), lambda i: (0, i, 0)),
            pl.BlockSpec((2, BR, D), lambda i: (0, i, 0)),
            pl.BlockSpec((2, D, D), lambda i: (0, 0, 0)),
            pl.BlockSpec((2, D, D), lambda i: (0, 0, 0)),
            pl.BlockSpec((2, 1, D), lambda i: (0, 0, 0)),
            pl.BlockSpec((D, D), lambda i: (0, 0)),
        ],
        out_specs=[
            pl.BlockSpec((2, BR, LANES), lambda i: (0, i, 0)),
            pl.BlockSpec((BR, LANES), lambda i: (i, 0)),
        ],
        out_shape=[
            jax.ShapeDtypeStruct((2, R1P, LANES), jnp.float32),
            jax.ShapeDtypeStruct((R1P, LANES), jnp.float32),
        ],
    )(hd, agg, deg, ws, wn, b2, wcat)


def _sc_layer1_body(q_ref, src_ref, dst_ref, sd_out,
                    src_v, dst_v, rows0_v, rows1_v, zb16_v, s_sp,
                    gsem0, gsem1, ssem0, ssem1):
    c = lax.axis_index("c")
    s = lax.axis_index("s")
    zv = jnp.zeros((LANES,), jnp.float32)

    def fill16(i, _):
        zb16_v[i, :] = zv
        return 0
    lax.fori_loop(0, CH, fill16, 0)

    base = s * (R2P // NS)
    pltpu.sync_copy(zb16_v, s_sp.at[pl.ds(base, CH)])
    pltpu.sync_copy(zb16_v, s_sp.at[pl.ds(base + CH, CH)])
    pltpu.sync_copy(zb16_v.at[:64], s_sp.at[pl.ds(base + 2 * CH, 64)])
    plsc.subcore_barrier()

    pltpu.sync_copy(src_ref.at[c, s], src_v)
    pltpu.sync_copy(dst_ref.at[c, s], dst_v)

    pltpu.async_copy(q_ref.at[src_v.at[0]], rows0_v, gsem0)
    pltpu.async_copy(q_ref.at[src_v.at[1]], rows1_v, gsem1)

    def pair(jj, _):
        j0 = 2 * jj
        pltpu.make_async_copy(q_ref.at[src_v.at[j0]], rows0_v, gsem0).wait()
        pltpu.async_copy(rows0_v, s_sp.at[dst_v.at[j0]], ssem0, add=True)
        pltpu.make_async_copy(q_ref.at[src_v.at[j0 + 1]], rows1_v,
                              gsem1).wait()
        pltpu.async_copy(rows1_v, s_sp.at[dst_v.at[j0 + 1]], ssem1, add=True)

        @pl.when(jj < C1 // 2 - 1)
        def _():
            pltpu.make_async_copy(rows0_v, s_sp.at[dst_v.at[j0]],
                                  ssem0).wait()
            pltpu.async_copy(q_ref.at[src_v.at[j0 + 2]], rows0_v, gsem0)
            pltpu.make_async_copy(rows1_v, s_sp.at[dst_v.at[j0 + 1]],
                                  ssem1).wait()
            pltpu.async_copy(q_ref.at[src_v.at[j0 + 3]], rows1_v, gsem1)
        return 0
    lax.fori_loop(0, C1 // 2, pair, 0)
    pltpu.make_async_copy(rows0_v, s_sp.at[dst_v.at[0]], ssem0).wait()
    pltpu.make_async_copy(rows1_v, s_sp.at[dst_v.at[1]], ssem1).wait()
    plsc.subcore_barrier()

    nrow = R2P // NS

    @pl.when(c == 0)
    def _():
        pltpu.sync_copy(s_sp.at[pl.ds(base, nrow)],
                        sd_out.at[pl.ds(base, nrow), pl.ds(0, LANES)])

    @pl.when(c == 1)
    def _():
        pltpu.sync_copy(s_sp.at[pl.ds(base, nrow)],
                        sd_out.at[pl.ds(base, nrow), pl.ds(LANES, LANES)])


_sc_layer1 = functools.partial(
    pl.kernel,
    out_type=jax.ShapeDtypeStruct((R2P, D), jnp.float32),
    mesh=plsc.VectorSubcoreMesh(core_axis_name="c", subcore_axis_name="s"),
    compiler_params=pltpu.CompilerParams(use_tc_tiling_on_sc=False),
    scratch_types=[
        pltpu.VMEM((C1, CH), jnp.int32),
        pltpu.VMEM((C1, CH), jnp.int32),
        pltpu.VMEM((CH, LANES), jnp.float32),
        pltpu.VMEM((CH, LANES), jnp.float32),
        pltpu.VMEM((CH, LANES), jnp.float32),
        pltpu.VMEM_SHARED((R2P, LANES), jnp.float32),
        pltpu.SemaphoreType.DMA,
        pltpu.SemaphoreType.DMA,
        pltpu.SemaphoreType.DMA,
        pltpu.SemaphoreType.DMA,
    ],
)(_sc_layer1_body)


def _tc_final_body(p_ref, sd_ref, dg_ref, b_ref, o_ref):
    p = p_ref[:, 0:1]
    sd = sd_ref[...]
    dg = dg_ref[...]
    s0 = sd[:, 0:1] / jnp.maximum(dg[0, :, 0:1], 1.0)
    s1 = sd[:, LANES:LANES + 1] / jnp.maximum(dg[1, :, 0:1], 1.0)
    o_ref[...] = jnp.broadcast_to(
        jax.nn.sigmoid(p + s0 + s1 + b_ref[0, 0:1]), (R2P, LANES))


def _tc_final(p, sd, deg1, bsum):
    return pl.pallas_call(
        _tc_final_body,
        grid=(1,),
        in_specs=[
            pl.BlockSpec((R2P, LANES), lambda i: (0, 0)),
            pl.BlockSpec((R2P, D), lambda i: (0, 0)),
            pl.BlockSpec((2, R2P, D), lambda i: (0, 0, 0)),
            pl.BlockSpec((1, LANES), lambda i: (0, 0)),
        ],
        out_specs=pl.BlockSpec((R2P, LANES), lambda i: (0, 0)),
        out_shape=jax.ShapeDtypeStruct((R2P, LANES), jnp.float32),
    )(p, sd, deg1, bsum)


def kernel(h, src0_r0, dst0_r0, src0_r1, dst0_r1, src1_r0, dst1_r0,
           src1_r1, dst1_r1, Wself0_r0, Wneigh0_r0, b0_r0, Wself0_r1,
           Wneigh0_r1, b0_r1, Wself1_r0, Wneigh1_r0, b1_r0, Wself1_r1,
           Wneigh1_r1, b1_r1):
    hflat = h.astype(jnp.bfloat16).reshape(2 * N0, DH)

    s0p, d0p = _pad_edges(src0_r0, dst0_r0, EP0, N0, N1, R1P)
    s1p, d1p = _pad_edges(src0_r1, dst0_r1, EP0, N0, N1, R1P)
    srcA = jnp.stack([
        jnp.stack([(2 * s0p).reshape(NS, C0, CH),
                   (2 * s1p).reshape(NS, C0, CH)]),
        jnp.stack([(2 * s0p + 1).reshape(NS, C0, CH),
                   (2 * s1p + 1).reshape(NS, C0, CH)]),
    ])
    dstA = jnp.stack([d0p.reshape(NS, C0, CH), d1p.reshape(NS, C0, CH)])

    agg = _sc_layer0(hflat, srcA, dstA)

    ws = jnp.stack([Wself0_r0, Wself0_r1])
    wn = jnp.stack([Wneigh0_r0, Wneigh0_r1])
    b2 = jnp.stack([b0_r0, b0_r1]).reshape(2, 1, D)
    wcat = jnp.concatenate(
        [Wneigh1_r0, Wneigh1_r1, Wself1_r0 + Wself1_r1,
         jnp.zeros((D, D - 3), jnp.float32)], axis=1)

    sb0, db0 = _pad_edges(src1_r0, dst1_r0, EP1, N1, N2, R2P)
    sb1, db1 = _pad_edges(src1_r1, dst1_r1, EP1, N1, N2, R2P)
    srcB = jnp.stack([sb0.reshape(NS, C1, CH),
                      (sb1 + R1P).reshape(NS, C1, CH)])
    dstB = jnp.stack([db0.reshape(NS, C1, CH), db1.reshape(NS, C1, CH)])

    deg0, deg1 = _sc_degrees(dstA, dstB)

    q2, p = _tc_dense(h, agg, deg0, ws, wn, b2, wcat)
    qf = q2.reshape(2 * R1P, LANES)

    sd = _sc_layer1(qf, srcB, dstB)

    bsum = jnp.broadcast_to((b1_r0 + b1_r1).reshape(1, 1), (1, LANES))
    out = _tc_final(p, sd, deg1, bsum)
    return out[:N2, 0:1]
